# Initial kernel scaffold; baseline (speedup 1.0000x reference)
import jax
import jax.numpy as jnp
from jax.experimental import pallas as pl


def kernel(x, fc_weight, bias):
    raise NotImplementedError("write your pallas kernel here")


# speedup vs baseline: 1.3428x; 1.3428x over previous
import functools

import jax
import jax.numpy as jnp
from jax import lax
from jax.experimental import pallas as pl
from jax.experimental.pallas import tpu as pltpu
from jax.experimental.pallas import tpu_sc as plsc

_B = 16384
_F = 26
_NW = 32
_BW = _B // _NW
_K = _F * _BW
_C = 128
_R = _K // _C


def _body(idx_hbm, table_hbm, out_hbm, idx_v, vals_v, out_v, sem):
    wid = lax.axis_index("s") * 2 + lax.axis_index("c")

    pltpu.sync_copy(idx_hbm.at[wid], idx_v)

    w = 16

    @pl.loop(0, _R)
    def _fire(j):
        pltpu.async_copy(table_hbm.at[idx_v.at[j]], vals_v.at[j], sem)

        @pl.when(j >= w)
        def _():
            pltpu.make_async_copy(
                table_hbm.at[idx_v.at[0]], vals_v.at[0], sem
            ).wait()

    @pl.loop(0, w)
    def _drain(j):
        pltpu.make_async_copy(
            table_hbm.at[idx_v.at[0]], vals_v.at[0], sem
        ).wait()

    for a in range(_BW // _C):
        for b in range(_C // 16):
            acc = vals_v[a, pl.ds(16 * b, 16)]
            for f in range(1, _F):
                acc = acc + vals_v[f * 4 + a, pl.ds(16 * b, 16)]
            out_v[pl.ds(128 * a + 16 * b, 16)] = acc

    pltpu.sync_copy(out_v, out_hbm.at[pl.ds(wid * _BW, _BW)])


@jax.jit
def _fl_kernel(idx_all, table):
    mesh = plsc.VectorSubcoreMesh(core_axis_name="c", subcore_axis_name="s")
    k = pl.kernel(
        _body,
        out_type=jax.ShapeDtypeStruct((_B,), jnp.float32),
        mesh=mesh,
        scratch_types=[
            pltpu.VMEM((_R, _C), jnp.int32),
            pltpu.VMEM((_R, _C), jnp.float32),
            pltpu.VMEM((_BW,), jnp.float32),
            pltpu.SemaphoreType.DMA,
        ],
    )
    return k(idx_all, table)


def kernel(x, fc_weight, bias):
    idx_all = (
        x.astype(jnp.int32)
        .T.reshape(_F, _NW, _BW)
        .transpose(1, 0, 2)
        .reshape(_NW, _R, _C)
    )
    table = fc_weight.reshape(-1)
    out = _fl_kernel(idx_all, table)
    return out.reshape(_B, 1) + bias[None, :]
